# baseline (speedup 1.0000x reference)
import functools

import jax
import jax.numpy as jnp
from jax import lax
from jax.experimental import pallas as pl
from jax.experimental.pallas import tpu as pltpu
from jax.experimental.pallas import tpu_sc as plsc

NC = 2
NS = 16
NW = NC * NS
F = 16
IDX_MINOR = 128
CH = 8


def _chunk_rows(rows_per_tile: int, cap: int) -> int:
    for d in range(min(cap, rows_per_tile), 7, -1):
        if rows_per_tile % d == 0 and d % 8 == 0:
            return d
    return 8


@functools.lru_cache(maxsize=None)
def _build_scatter(num_nodes: int, total_windows: int):
    wq, wr = divmod(total_windows, NW)
    epc = CH * IDX_MINOR
    rows_per_tile = num_nodes // NS
    zrows = _chunk_rows(rows_per_tile, 512)
    n_z = rows_per_tile // zrows

    mesh = plsc.VectorSubcoreMesh(core_axis_name="c", subcore_axis_name="s")

    @functools.partial(
        pl.kernel,
        mesh=mesh,
        compiler_params=pltpu.CompilerParams(use_tc_tiling_on_sc=False),
        out_type=jax.ShapeDtypeStruct((NC * num_nodes, F), jnp.float32),
        scratch_types=[
            pltpu.VMEM((epc, F), jnp.float32),
            pltpu.VMEM((CH, IDX_MINOR), jnp.int32),
            pltpu.VMEM((zrows, F), jnp.float32),
            pltpu.VMEM_SHARED((num_nodes, F), jnp.float32),
        ],
    )
    def scatter_partials(edges_hbm, recv_hbm, out_hbm, ebuf, ibuf, dbuf, acc):
        c = lax.axis_index("c")
        s = lax.axis_index("s")
        wid = c * NS + s

        def zero_body(i, carry):
            dbuf[i, :] = jnp.zeros((F,), jnp.float32)
            return carry

        lax.fori_loop(0, zrows, zero_body, 0)
        for z in range(n_z):
            pltpu.sync_copy(
                dbuf, acc.at[pl.ds(s * rows_per_tile + z * zrows, zrows)]
            )
        plsc.subcore_barrier()

        steps = wq + jnp.where(wid < wr, 1, 0)
        win_base = wq * wid + jnp.minimum(wid, wr)

        def step(t, carry):
            i0 = (win_base + t) * CH * IDX_MINOR
            pltpu.sync_copy(edges_hbm.at[pl.ds(i0, epc)], ebuf)
            for j in range(CH):
                pltpu.sync_copy(
                    recv_hbm.at[pl.ds(i0 + j * IDX_MINOR, IDX_MINOR)],
                    ibuf.at[j],
                )
            for j in range(CH):
                pltpu.sync_copy(
                    ebuf.at[pl.ds(j * IDX_MINOR, IDX_MINOR)],
                    acc.at[ibuf.at[j]],
                    add=True,
                )
            return carry

        lax.fori_loop(0, steps, step, 0)
        plsc.subcore_barrier()

        out_base = c * num_nodes + s * rows_per_tile
        for z in range(n_z):
            pltpu.sync_copy(
                acc.at[pl.ds(s * rows_per_tile + z * zrows, zrows)], dbuf
            )
            pltpu.sync_copy(dbuf, out_hbm.at[pl.ds(out_base + z * zrows, zrows)])

    return scatter_partials


def _merge_body(a_ref, b_ref, o_ref):
    o_ref[...] = a_ref[...] + b_ref[...]


def _merge(partials, num_nodes, n_pad):
    br = _chunk_rows(n_pad, 4096)
    nblk = -(-num_nodes // br)
    return pl.pallas_call(
        _merge_body,
        out_shape=jax.ShapeDtypeStruct((num_nodes, F), jnp.float32),
        grid=(nblk,),
        in_specs=[
            pl.BlockSpec((br, F), lambda i: (i, 0)),
            pl.BlockSpec((br, F), lambda i: (i + n_pad // br, 0)),
        ],
        out_specs=pl.BlockSpec((br, F), lambda i: (i, 0)),
    )(partials, partials)


def kernel(nodes, edges, receivers):
    num_nodes = nodes.shape[0]
    num_edges, f = edges.shape

    step_edges = IDX_MINOR * CH
    e_pad = -(-num_edges // step_edges) * step_edges
    pad = e_pad - num_edges
    recv = receivers.astype(jnp.int32)
    if pad:
        edges = jnp.concatenate(
            [edges, jnp.zeros((pad, f), edges.dtype)], axis=0
        )
        recv = jnp.concatenate(
            [recv, jnp.arange(pad, dtype=jnp.int32) % num_nodes], axis=0
        )

    n_align = NS * 32
    n_pad = -(-num_nodes // n_align) * n_align
    total_windows = e_pad // step_edges
    partials = _build_scatter(n_pad, total_windows)(edges, recv)
    return _merge(partials, num_nodes, n_pad)


# speedup vs baseline: 1.2282x; 1.2282x over previous
import functools

import jax
import jax.numpy as jnp
from jax import lax
from jax.experimental import pallas as pl
from jax.experimental.pallas import tpu as pltpu
from jax.experimental.pallas import tpu_sc as plsc

NC = 2
NS = 16
NW = NC * NS
F = 16
IDX_MINOR = 128
CH = 8


def _chunk_rows(rows_per_tile: int, cap: int) -> int:
    for d in range(min(cap, rows_per_tile), 7, -1):
        if rows_per_tile % d == 0 and d % 8 == 0:
            return d
    return 8


@functools.lru_cache(maxsize=None)
def _build_scatter(num_nodes: int, total_windows: int):
    wq, wr = divmod(total_windows, NW)
    epc = CH * IDX_MINOR
    rows_per_tile = num_nodes // NS
    zrows = _chunk_rows(rows_per_tile, 512)
    n_z = rows_per_tile // zrows

    mesh = plsc.VectorSubcoreMesh(core_axis_name="c", subcore_axis_name="s")

    @functools.partial(
        pl.kernel,
        mesh=mesh,
        compiler_params=pltpu.CompilerParams(use_tc_tiling_on_sc=False),
        out_type=jax.ShapeDtypeStruct((NC * num_nodes, F), jnp.float32),
        scratch_types=[
            pltpu.VMEM((CH, IDX_MINOR, F), jnp.float32),
            pltpu.VMEM((CH, IDX_MINOR), jnp.int32),
            pltpu.VMEM((zrows, F), jnp.float32),
            pltpu.VMEM_SHARED((num_nodes, F), jnp.float32),
            pltpu.SemaphoreType.DMA,
        ],
    )
    def scatter_partials(
        edges_hbm, recv_hbm, out_hbm, ebuf, ibuf, dbuf, acc, sem
    ):
        c = lax.axis_index("c")
        s = lax.axis_index("s")
        wid = c * NS + s

        def zero_body(i, carry):
            dbuf[i, :] = jnp.zeros((F,), jnp.float32)
            return carry

        lax.fori_loop(0, zrows, zero_body, 0)
        for z in range(n_z):
            pltpu.sync_copy(
                dbuf, acc.at[pl.ds(s * rows_per_tile + z * zrows, zrows)]
            )
        plsc.subcore_barrier()

        steps = wq + jnp.where(wid < wr, 1, 0)
        win_base = wq * wid + jnp.minimum(wid, wr)

        def step(t, carry):
            b0 = (win_base + t) * CH
            cp0 = pltpu.async_copy(edges_hbm.at[pl.ds(b0, CH)], ebuf, sem)
            cp1 = pltpu.async_copy(recv_hbm.at[pl.ds(b0, CH)], ibuf, sem)
            cp0.wait()
            cp1.wait()
            for j in range(CH):
                pltpu.sync_copy(
                    ebuf.at[j],
                    acc.at[ibuf.at[j]],
                    add=True,
                )
            return carry

        lax.fori_loop(0, steps, step, 0)
        plsc.subcore_barrier()

        out_base = c * num_nodes + s * rows_per_tile
        for z in range(n_z):
            pltpu.sync_copy(
                acc.at[pl.ds(s * rows_per_tile + z * zrows, zrows)], dbuf
            )
            pltpu.sync_copy(dbuf, out_hbm.at[pl.ds(out_base + z * zrows, zrows)])

    return scatter_partials


def _merge_body(a_ref, b_ref, o_ref):
    o_ref[...] = a_ref[...] + b_ref[...]


def _merge(partials, num_nodes, n_pad):
    br = _chunk_rows(n_pad, 4096)
    nblk = -(-num_nodes // br)
    return pl.pallas_call(
        _merge_body,
        out_shape=jax.ShapeDtypeStruct((num_nodes, F), jnp.float32),
        grid=(nblk,),
        in_specs=[
            pl.BlockSpec((br, F), lambda i: (i, 0)),
            pl.BlockSpec((br, F), lambda i: (i + n_pad // br, 0)),
        ],
        out_specs=pl.BlockSpec((br, F), lambda i: (i, 0)),
    )(partials, partials)


def kernel(nodes, edges, receivers):
    num_nodes = nodes.shape[0]
    num_edges, f = edges.shape

    step_edges = IDX_MINOR * CH
    e_pad = -(-num_edges // step_edges) * step_edges
    pad = e_pad - num_edges
    recv = receivers.astype(jnp.int32)
    if pad:
        edges = jnp.concatenate(
            [edges, jnp.zeros((pad, f), edges.dtype)], axis=0
        )
        recv = jnp.concatenate(
            [recv, jnp.arange(pad, dtype=jnp.int32) % num_nodes], axis=0
        )

    n_align = NS * 32
    n_pad = -(-num_nodes // n_align) * n_align
    total_windows = e_pad // step_edges
    partials = _build_scatter(n_pad, total_windows)(
        edges.reshape(e_pad // IDX_MINOR, IDX_MINOR, f),
        recv.reshape(e_pad // IDX_MINOR, IDX_MINOR),
    )
    return _merge(partials, num_nodes, n_pad)


# speedup vs baseline: 1.2843x; 1.0456x over previous
import functools

import jax
import jax.numpy as jnp
from jax import lax
from jax.experimental import pallas as pl
from jax.experimental.pallas import tpu as pltpu
from jax.experimental.pallas import tpu_sc as plsc

NC = 2
NS = 16
NW = NC * NS
F = 16
IDX_MINOR = 128
CH = 8


def _chunk_rows(rows_per_tile: int, cap: int) -> int:
    for d in range(min(cap, rows_per_tile), 7, -1):
        if rows_per_tile % d == 0 and d % 8 == 0:
            return d
    return 8


@functools.lru_cache(maxsize=None)
def _build_scatter(num_nodes: int, total_windows: int):
    wq, wr = divmod(total_windows, NW)
    epc = CH * IDX_MINOR
    rows_per_tile = num_nodes // NS
    zrows = _chunk_rows(rows_per_tile, 512)
    n_z = rows_per_tile // zrows

    mesh = plsc.VectorSubcoreMesh(core_axis_name="c", subcore_axis_name="s")

    @functools.partial(
        pl.kernel,
        mesh=mesh,
        compiler_params=pltpu.CompilerParams(use_tc_tiling_on_sc=False),
        out_type=jax.ShapeDtypeStruct((NC * num_nodes, F), jnp.float32),
        scratch_types=[
            pltpu.VMEM((CH, IDX_MINOR, F), jnp.float32),
            pltpu.VMEM((CH, IDX_MINOR), jnp.int32),
            pltpu.VMEM((zrows, F), jnp.float32),
            pltpu.VMEM_SHARED((num_nodes, F), jnp.float32),
            pltpu.SemaphoreType.DMA,
        ],
    )
    def scatter_partials(
        edges_hbm, recv_hbm, out_hbm, ebuf, ibuf, dbuf, acc, sem
    ):
        c = lax.axis_index("c")
        s = lax.axis_index("s")
        wid = c * NS + s

        def zero_body(i, carry):
            dbuf[i, :] = jnp.zeros((F,), jnp.float32)
            return carry

        lax.fori_loop(0, zrows, zero_body, 0)
        for z in range(n_z):
            pltpu.sync_copy(
                dbuf, acc.at[pl.ds(s * rows_per_tile + z * zrows, zrows)]
            )
        plsc.subcore_barrier()

        steps = wq + jnp.where(wid < wr, 1, 0)
        win_base = wq * wid + jnp.minimum(wid, wr)

        def step(t, carry):
            b0 = (win_base + t) * CH
            cp0 = pltpu.async_copy(edges_hbm.at[pl.ds(b0, CH)], ebuf, sem)
            cp1 = pltpu.async_copy(recv_hbm.at[pl.ds(b0, CH)], ibuf, sem)
            cp0.wait()
            cp1.wait()
            for j in range(CH):
                pltpu.sync_copy(
                    ebuf.at[j],
                    acc.at[ibuf.at[j]],
                    add=True,
                )
            return carry

        lax.fori_loop(0, steps, step, 0)
        plsc.subcore_barrier()

        out_base = c * num_nodes + s * rows_per_tile
        for z in range(n_z):
            pltpu.sync_copy(
                acc.at[pl.ds(s * rows_per_tile + z * zrows, zrows)], dbuf
            )
            pltpu.sync_copy(dbuf, out_hbm.at[pl.ds(out_base + z * zrows, zrows)])

    return scatter_partials


@functools.lru_cache(maxsize=None)
def _build_sc_merge(num_nodes: int, n_pad: int):
    rows = -(-num_nodes // (NW * 8)) * 8
    mesh = plsc.VectorSubcoreMesh(core_axis_name="c", subcore_axis_name="s")

    @functools.partial(
        pl.kernel,
        mesh=mesh,
        compiler_params=pltpu.CompilerParams(use_tc_tiling_on_sc=False),
        out_type=jax.ShapeDtypeStruct((num_nodes, F), jnp.float32),
        scratch_types=[
            pltpu.VMEM((rows, F), jnp.float32),
            pltpu.VMEM((rows, F), jnp.float32),
            pltpu.SemaphoreType.DMA,
        ],
    )
    def merge(p_hbm, out_hbm, abuf, bbuf, sem):
        c = lax.axis_index("c")
        s = lax.axis_index("s")
        wid = c * NS + s
        start = jnp.minimum(wid * rows, num_nodes - rows)
        cp0 = pltpu.async_copy(p_hbm.at[pl.ds(start, rows)], abuf, sem)
        cp1 = pltpu.async_copy(p_hbm.at[pl.ds(n_pad + start, rows)], bbuf, sem)
        cp0.wait()
        cp1.wait()

        def add_body(i, carry):
            abuf[i, :] = abuf[i, :] + bbuf[i, :]
            return carry

        lax.fori_loop(0, rows, add_body, 0)
        pltpu.sync_copy(abuf, out_hbm.at[pl.ds(start, rows)])

    return merge


def _merge_body(a_ref, b_ref, o_ref):
    o_ref[...] = a_ref[...] + b_ref[...]


def _merge(partials, num_nodes, n_pad):
    br = _chunk_rows(n_pad, 4096)
    nblk = -(-num_nodes // br)
    return pl.pallas_call(
        _merge_body,
        out_shape=jax.ShapeDtypeStruct((num_nodes, F), jnp.float32),
        grid=(nblk,),
        in_specs=[
            pl.BlockSpec((br, F), lambda i: (i, 0)),
            pl.BlockSpec((br, F), lambda i: (i + n_pad // br, 0)),
        ],
        out_specs=pl.BlockSpec((br, F), lambda i: (i, 0)),
    )(partials, partials)


def kernel(nodes, edges, receivers):
    num_nodes = nodes.shape[0]
    num_edges, f = edges.shape

    step_edges = IDX_MINOR * CH
    e_pad = -(-num_edges // step_edges) * step_edges
    pad = e_pad - num_edges
    recv = receivers.astype(jnp.int32)
    if pad:
        edges = jnp.concatenate(
            [edges, jnp.zeros((pad, f), edges.dtype)], axis=0
        )
        recv = jnp.concatenate(
            [recv, jnp.arange(pad, dtype=jnp.int32) % num_nodes], axis=0
        )

    n_align = NS * 32
    n_pad = -(-num_nodes // n_align) * n_align
    total_windows = e_pad // step_edges
    partials = _build_scatter(n_pad, total_windows)(
        edges.reshape(e_pad // IDX_MINOR, IDX_MINOR, f),
        recv.reshape(e_pad // IDX_MINOR, IDX_MINOR),
    )
    return _build_sc_merge(num_nodes, n_pad)(partials)


# speedup vs baseline: 1.3760x; 1.0715x over previous
import functools

import jax
import jax.numpy as jnp
from jax import lax
from jax.experimental import pallas as pl
from jax.experimental.pallas import tpu as pltpu
from jax.experimental.pallas import tpu_sc as plsc

NC = 2
NS = 16
NW = NC * NS
F = 16
IDX_MINOR = 128
CH = 4
NBUF = 2


def _chunk_rows(rows_per_tile: int, cap: int) -> int:
    for d in range(min(cap, rows_per_tile), 7, -1):
        if rows_per_tile % d == 0 and d % 8 == 0:
            return d
    return 8


@functools.lru_cache(maxsize=None)
def _build_scatter(num_nodes: int, total_windows: int):
    wq, wr = divmod(total_windows, NW)
    epc = CH * IDX_MINOR
    rows_per_tile = num_nodes // NS
    zrows = _chunk_rows(rows_per_tile, 512)
    n_z = rows_per_tile // zrows

    mesh = plsc.VectorSubcoreMesh(core_axis_name="c", subcore_axis_name="s")

    @functools.partial(
        pl.kernel,
        mesh=mesh,
        compiler_params=pltpu.CompilerParams(use_tc_tiling_on_sc=False),
        out_type=jax.ShapeDtypeStruct((NC * num_nodes, F), jnp.float32),
        scratch_types=[
            pltpu.VMEM((NBUF, CH, IDX_MINOR, F), jnp.float32),
            pltpu.VMEM((NBUF, CH, IDX_MINOR), jnp.int32),
            pltpu.VMEM((zrows, F), jnp.float32),
            pltpu.VMEM_SHARED((num_nodes, F), jnp.float32),
            pltpu.SemaphoreType.DMA,
            pltpu.SemaphoreType.DMA,
        ],
    )
    def scatter_partials(
        edges_hbm, recv_hbm, out_hbm, ebuf, ibuf, dbuf, acc, lsem, ssem
    ):
        c = lax.axis_index("c")
        s = lax.axis_index("s")
        wid = c * NS + s

        def zero_body(i, carry):
            dbuf[i, :] = jnp.zeros((F,), jnp.float32)
            return carry

        lax.fori_loop(0, zrows, zero_body, 0)
        for z in range(n_z):
            pltpu.sync_copy(
                dbuf, acc.at[pl.ds(s * rows_per_tile + z * zrows, zrows)]
            )
        plsc.subcore_barrier()

        steps = wq + jnp.where(wid < wr, 1, 0)
        win_base = wq * wid + jnp.minimum(wid, wr)

        def fire_loads(t, k):
            b0 = (win_base + t) * CH
            pltpu.async_copy(edges_hbm.at[pl.ds(b0, CH)], ebuf.at[k], lsem)
            pltpu.async_copy(recv_hbm.at[pl.ds(b0, CH)], ibuf.at[k], lsem)

        def wait_loads(k):
            pltpu.make_async_copy(
                edges_hbm.at[pl.ds(0, CH)], ebuf.at[k], lsem
            ).wait()
            pltpu.make_async_copy(
                recv_hbm.at[pl.ds(0, CH)], ibuf.at[k], lsem
            ).wait()

        def drain_scatters(k):
            for j in range(CH):
                pltpu.make_async_copy(
                    ebuf.at[k].at[j], acc.at[ibuf.at[k].at[j]], ssem
                ).wait()

        @pl.when(steps > 0)
        def _():
            fire_loads(0, 0)

        def step(t, carry):
            k = lax.rem(t, 2)
            kn = lax.rem(t + 1, 2)

            @pl.when(t > 0)
            def _():
                drain_scatters(kn)

            @pl.when(t + 1 < steps)
            def _():
                fire_loads(t + 1, kn)

            wait_loads(k)
            for j in range(CH):
                pltpu.async_copy(
                    ebuf.at[k].at[j], acc.at[ibuf.at[k].at[j]], ssem,
                    add=True,
                )
            return carry

        lax.fori_loop(0, steps, step, 0)

        @pl.when(steps > 0)
        def _():
            drain_scatters(lax.rem(steps - 1, 2))

        plsc.subcore_barrier()

        out_base = c * num_nodes + s * rows_per_tile
        for z in range(n_z):
            pltpu.sync_copy(
                acc.at[pl.ds(s * rows_per_tile + z * zrows, zrows)], dbuf
            )
            pltpu.sync_copy(dbuf, out_hbm.at[pl.ds(out_base + z * zrows, zrows)])

    return scatter_partials


@functools.lru_cache(maxsize=None)
def _build_sc_merge(num_nodes: int, n_pad: int):
    rows = -(-num_nodes // (NW * 8)) * 8
    mesh = plsc.VectorSubcoreMesh(core_axis_name="c", subcore_axis_name="s")

    @functools.partial(
        pl.kernel,
        mesh=mesh,
        compiler_params=pltpu.CompilerParams(use_tc_tiling_on_sc=False),
        out_type=jax.ShapeDtypeStruct((num_nodes, F), jnp.float32),
        scratch_types=[
            pltpu.VMEM((rows, F), jnp.float32),
            pltpu.VMEM((rows, F), jnp.float32),
            pltpu.SemaphoreType.DMA,
        ],
    )
    def merge(p_hbm, out_hbm, abuf, bbuf, sem):
        c = lax.axis_index("c")
        s = lax.axis_index("s")
        wid = c * NS + s
        start = jnp.minimum(wid * rows, num_nodes - rows)
        cp0 = pltpu.async_copy(p_hbm.at[pl.ds(start, rows)], abuf, sem)
        cp1 = pltpu.async_copy(p_hbm.at[pl.ds(n_pad + start, rows)], bbuf, sem)
        cp0.wait()
        cp1.wait()

        def add_body(i, carry):
            abuf[i, :] = abuf[i, :] + bbuf[i, :]
            return carry

        lax.fori_loop(0, rows, add_body, 0)
        pltpu.sync_copy(abuf, out_hbm.at[pl.ds(start, rows)])

    return merge


def _merge_body(a_ref, b_ref, o_ref):
    o_ref[...] = a_ref[...] + b_ref[...]


def _merge(partials, num_nodes, n_pad):
    br = _chunk_rows(n_pad, 4096)
    nblk = -(-num_nodes // br)
    return pl.pallas_call(
        _merge_body,
        out_shape=jax.ShapeDtypeStruct((num_nodes, F), jnp.float32),
        grid=(nblk,),
        in_specs=[
            pl.BlockSpec((br, F), lambda i: (i, 0)),
            pl.BlockSpec((br, F), lambda i: (i + n_pad // br, 0)),
        ],
        out_specs=pl.BlockSpec((br, F), lambda i: (i, 0)),
    )(partials, partials)


def kernel(nodes, edges, receivers):
    num_nodes = nodes.shape[0]
    num_edges, f = edges.shape

    step_edges = IDX_MINOR * CH
    e_pad = -(-num_edges // step_edges) * step_edges
    pad = e_pad - num_edges
    recv = receivers.astype(jnp.int32)
    if pad:
        edges = jnp.concatenate(
            [edges, jnp.zeros((pad, f), edges.dtype)], axis=0
        )
        recv = jnp.concatenate(
            [recv, jnp.arange(pad, dtype=jnp.int32) % num_nodes], axis=0
        )

    n_align = NS * 32
    n_pad = -(-num_nodes // n_align) * n_align
    total_windows = e_pad // step_edges
    partials = _build_scatter(n_pad, total_windows)(
        edges.reshape(e_pad // IDX_MINOR, IDX_MINOR, f),
        recv.reshape(e_pad // IDX_MINOR, IDX_MINOR),
    )
    return _build_sc_merge(num_nodes, n_pad)(partials)


# speedup vs baseline: 1.3766x; 1.0004x over previous
import functools

import jax
import jax.numpy as jnp
from jax import lax
from jax.experimental import pallas as pl
from jax.experimental.pallas import tpu as pltpu
from jax.experimental.pallas import tpu_sc as plsc

NC = 2
NS = 16
NW = NC * NS
F = 16
IDX_MINOR = 128
CH = 4
NBUF = 2


def _chunk_rows(rows_per_tile: int, cap: int) -> int:
    for d in range(min(cap, rows_per_tile), 7, -1):
        if rows_per_tile % d == 0 and d % 8 == 0:
            return d
    return 8


@functools.lru_cache(maxsize=None)
def _build_scatter(num_nodes: int, total_windows: int):
    wq, wr = divmod(total_windows, NW)
    epc = CH * IDX_MINOR
    rows_per_tile = num_nodes // NS
    zrows = _chunk_rows(rows_per_tile, 512)
    n_z = rows_per_tile // zrows

    mesh = plsc.VectorSubcoreMesh(core_axis_name="c", subcore_axis_name="s")

    @functools.partial(
        pl.kernel,
        mesh=mesh,
        compiler_params=pltpu.CompilerParams(use_tc_tiling_on_sc=False),
        out_type=jax.ShapeDtypeStruct((NC * num_nodes, F), jnp.float32),
        scratch_types=[
            pltpu.VMEM((NBUF, CH, IDX_MINOR, F), jnp.float32),
            pltpu.VMEM((NBUF, CH, IDX_MINOR), jnp.int32),
            pltpu.VMEM((zrows, F), jnp.float32),
            pltpu.VMEM_SHARED((num_nodes, F), jnp.float32),
            pltpu.SemaphoreType.DMA,
            pltpu.SemaphoreType.DMA,
        ],
    )
    def scatter_partials(
        edges_hbm, recv_hbm, out_hbm, ebuf, ibuf, dbuf, acc, lsem, ssem
    ):
        c = lax.axis_index("c")
        s = lax.axis_index("s")
        wid = c * NS + s

        def zero_body(i, carry):
            dbuf[i, :] = jnp.zeros((F,), jnp.float32)
            return carry

        lax.fori_loop(0, zrows, zero_body, 0)
        for z in range(n_z):
            pltpu.sync_copy(
                dbuf, acc.at[pl.ds(s * rows_per_tile + z * zrows, zrows)]
            )
        plsc.subcore_barrier()

        steps = wq + jnp.where(wid < wr, 1, 0)
        win_base = wq * wid + jnp.minimum(wid, wr)

        def fire_loads(t, k):
            b0 = (win_base + t) * CH
            pltpu.async_copy(edges_hbm.at[pl.ds(b0, CH)], ebuf.at[k], lsem)
            pltpu.async_copy(recv_hbm.at[pl.ds(b0, CH)], ibuf.at[k], lsem)

        def wait_loads(k):
            pltpu.make_async_copy(
                edges_hbm.at[pl.ds(0, CH)], ebuf.at[k], lsem
            ).wait()
            pltpu.make_async_copy(
                recv_hbm.at[pl.ds(0, CH)], ibuf.at[k], lsem
            ).wait()

        def drain_scatters(k):
            for j in range(CH):
                pltpu.make_async_copy(
                    ebuf.at[k].at[j], acc.at[ibuf.at[k].at[j]], ssem
                ).wait()

        @pl.when(steps > 0)
        def _():
            fire_loads(0, 0)

        def step(t, carry):
            k = lax.rem(t, 2)
            kn = lax.rem(t + 1, 2)

            @pl.when(t > 0)
            def _():
                drain_scatters(kn)

            @pl.when(t + 1 < steps)
            def _():
                fire_loads(t + 1, kn)

            wait_loads(k)
            for j in range(CH):
                pltpu.async_copy(
                    ebuf.at[k].at[j], acc.at[ibuf.at[k].at[j]], ssem,
                    add=True,
                )
            return carry

        lax.fori_loop(0, steps, step, 0)

        @pl.when(steps > 0)
        def _():
            drain_scatters(lax.rem(steps - 1, 2))

        plsc.subcore_barrier()

        out_base = c * num_nodes + s * rows_per_tile
        for z in range(n_z):
            pltpu.sync_copy(
                acc.at[pl.ds(s * rows_per_tile + z * zrows, zrows)], dbuf
            )
            pltpu.sync_copy(dbuf, out_hbm.at[pl.ds(out_base + z * zrows, zrows)])

    return scatter_partials


@functools.lru_cache(maxsize=None)
def _build_sc_merge(num_nodes: int, n_pad: int):
    rows = -(-num_nodes // (NW * 8)) * 8
    mesh = plsc.VectorSubcoreMesh(core_axis_name="c", subcore_axis_name="s")

    @functools.partial(
        pl.kernel,
        mesh=mesh,
        compiler_params=pltpu.CompilerParams(use_tc_tiling_on_sc=False),
        out_type=jax.ShapeDtypeStruct((num_nodes, F), jnp.float32),
        scratch_types=[
            pltpu.VMEM((rows, F), jnp.float32),
            pltpu.VMEM((rows, F), jnp.float32),
            pltpu.SemaphoreType.DMA,
        ],
    )
    def merge(p_hbm, out_hbm, abuf, bbuf, sem):
        c = lax.axis_index("c")
        s = lax.axis_index("s")
        wid = c * NS + s
        start = jnp.minimum(wid * rows, num_nodes - rows)
        cp0 = pltpu.async_copy(p_hbm.at[pl.ds(start, rows)], abuf, sem)
        cp1 = pltpu.async_copy(p_hbm.at[pl.ds(n_pad + start, rows)], bbuf, sem)
        cp0.wait()
        cp1.wait()

        def add_body(i, carry):
            abuf[i, :] = abuf[i, :] + bbuf[i, :]
            return carry

        lax.fori_loop(0, rows, add_body, 0)
        pltpu.sync_copy(abuf, out_hbm.at[pl.ds(start, rows)])

    return merge


def _merge_body(a_ref, b_ref, o_ref):
    o_ref[...] = a_ref[...] + b_ref[...]


def _merge(partials, num_nodes, n_pad):
    br = _chunk_rows(n_pad, 4096)
    nblk = -(-num_nodes // br)
    return pl.pallas_call(
        _merge_body,
        out_shape=jax.ShapeDtypeStruct((num_nodes, F), jnp.float32),
        grid=(nblk,),
        in_specs=[
            pl.BlockSpec((br, F), lambda i: (i, 0)),
            pl.BlockSpec((br, F), lambda i: (i + n_pad // br, 0)),
        ],
        out_specs=pl.BlockSpec((br, F), lambda i: (i, 0)),
    )(partials, partials)


def kernel(nodes, edges, receivers):
    num_nodes = nodes.shape[0]
    num_edges, f = edges.shape

    step_edges = IDX_MINOR * CH
    e_pad = -(-num_edges // step_edges) * step_edges
    pad = e_pad - num_edges
    recv = receivers.astype(jnp.int32)
    if pad:
        edges = jnp.concatenate(
            [edges, jnp.zeros((pad, f), edges.dtype)], axis=0
        )
        recv = jnp.concatenate(
            [recv, jnp.arange(pad, dtype=jnp.int32) % num_nodes], axis=0
        )

    n_align = NS * 32
    n_pad = -(-num_nodes // n_align) * n_align
    total_windows = e_pad // step_edges
    e128 = lax.optimization_barrier(edges.reshape(e_pad * f // 128, 128))
    partials = _build_scatter(n_pad, total_windows)(
        e128.reshape(e_pad // IDX_MINOR, IDX_MINOR, f),
        recv.reshape(e_pad // IDX_MINOR, IDX_MINOR),
    )
    return _build_sc_merge(num_nodes, n_pad)(partials)
